# baseline (speedup 1.0000x reference)
import functools

import jax
import jax.numpy as jnp
from jax import lax
from jax.experimental import pallas as pl
from jax.experimental.pallas import tpu as pltpu
from jax.experimental.pallas import tpu_sc as plsc

VOCAB = 100000
D_MODEL = 1024
BATCH = 4
SEQ_LEN = 2048

NUM_CORES = 2
NUM_SUBCORES = 16
NW = NUM_CORES * NUM_SUBCORES
B_TOTAL = BATCH * SEQ_LEN
B_PER_W = B_TOTAL // NW
CHUNK = 16
NCHUNK = B_PER_W // CHUNK
NBUF = 7
DEPTH = 4


def _emb_body(tok_hbm, table_hbm, out_hbm, idx_v, *rest):
    bufs = rest[:NBUF]
    gsem, ssem = rest[NBUF:]
    wid = lax.axis_index("s") * NUM_CORES + lax.axis_index("c")
    base = wid * B_PER_W
    pltpu.sync_copy(tok_hbm.at[wid], idx_v)

    def gather(c):
        cp = pltpu.make_async_copy(table_hbm.at[idx_v.at[c]], bufs[c % NBUF], gsem)
        cp.start()
        return cp

    def scatter(c):
        cp = pltpu.make_async_copy(
            bufs[c % NBUF], out_hbm.at[pl.ds(base + c * CHUNK, CHUNK)], ssem)
        cp.start()
        return cp

    gat = [None] * NCHUNK
    sca = [None] * NCHUNK
    for c in range(DEPTH):
        gat[c] = gather(c)
    for c in range(NCHUNK):
        gat[c].wait()
        sca[c] = scatter(c)
        nxt = c + DEPTH
        if nxt < NCHUNK:
            if nxt >= NBUF:
                sca[nxt - NBUF].wait()
            gat[nxt] = gather(nxt)
    for c in range(NCHUNK - NBUF, NCHUNK):
        sca[c].wait()


@jax.jit
def _embedding_lookup(tokens_grouped, table):
    mesh = plsc.VectorSubcoreMesh(core_axis_name="c", subcore_axis_name="s")
    run = pl.kernel(
        _emb_body,
        out_type=jax.ShapeDtypeStruct((B_TOTAL, D_MODEL), jnp.float32),
        mesh=mesh,
        scratch_types=(
            [pltpu.VMEM((NCHUNK, CHUNK), jnp.int32)]
            + [pltpu.VMEM((CHUNK, D_MODEL), jnp.float32) for _ in range(NBUF)]
            + [pltpu.SemaphoreType.DMA, pltpu.SemaphoreType.DMA]
        ),
    )
    return run(tokens_grouped, table)


def kernel(tokens, embedding_weight):
    tok = tokens.astype(jnp.int32).reshape(NW, NCHUNK, CHUNK)
    out = _embedding_lookup(tok, embedding_weight)
    return out.reshape(BATCH, SEQ_LEN, D_MODEL)


# speedup vs baseline: 1.0046x; 1.0046x over previous
import jax
import jax.numpy as jnp
from jax import lax
from jax.experimental import pallas as pl
from jax.experimental.pallas import tpu as pltpu
from jax.experimental.pallas import tpu_sc as plsc

VOCAB = 100000
D_MODEL = 1024
BATCH = 4
SEQ_LEN = 2048

NUM_CORES = 2
NUM_SUBCORES = 16
NW = NUM_CORES * NUM_SUBCORES
B_TOTAL = BATCH * SEQ_LEN
B_PER_W = B_TOTAL // NW
W_PER_B = NW // BATCH
CHUNK = 32
NCHUNK = B_PER_W // CHUNK
NBUF = 3
DEPTH = 2


def _emb_body(tok_hbm, table_hbm, out_hbm, idx_v, *rest):
    bufs = rest[:NBUF]
    gsem, ssem = rest[NBUF:]
    wid = lax.axis_index("s") * NUM_CORES + lax.axis_index("c")
    b = wid // W_PER_B
    off = (wid % W_PER_B) * B_PER_W
    pltpu.sync_copy(tok_hbm.at[b, pl.ds(off, B_PER_W)], idx_v)

    def gather(c):
        cp = pltpu.make_async_copy(
            table_hbm.at[idx_v.at[pl.ds(c * CHUNK, CHUNK)]], bufs[c % NBUF], gsem)
        cp.start()
        return cp

    def scatter(c):
        cp = pltpu.make_async_copy(
            bufs[c % NBUF], out_hbm.at[b, pl.ds(off + c * CHUNK, CHUNK)], ssem)
        cp.start()
        return cp

    gat = [None] * NCHUNK
    sca = [None] * NCHUNK
    for c in range(DEPTH):
        gat[c] = gather(c)
    for c in range(NCHUNK):
        gat[c].wait()
        sca[c] = scatter(c)
        nxt = c + DEPTH
        if nxt < NCHUNK:
            if nxt >= NBUF:
                sca[nxt - NBUF].wait()
            gat[nxt] = gather(nxt)
    for c in range(max(NCHUNK - NBUF, 0), NCHUNK):
        sca[c].wait()


@jax.jit
def _embedding_lookup(tokens, table):
    mesh = plsc.VectorSubcoreMesh(core_axis_name="c", subcore_axis_name="s")
    run = pl.kernel(
        _emb_body,
        out_type=jax.ShapeDtypeStruct((BATCH, SEQ_LEN, D_MODEL), jnp.float32),
        mesh=mesh,
        scratch_types=(
            [pltpu.VMEM((B_PER_W,), jnp.int32)]
            + [pltpu.VMEM((CHUNK, D_MODEL), jnp.float32) for _ in range(NBUF)]
            + [pltpu.SemaphoreType.DMA, pltpu.SemaphoreType.DMA]
        ),
    )
    return run(tokens, table)


def kernel(tokens, embedding_weight):
    return _embedding_lookup(tokens.astype(jnp.int32), embedding_weight)


# speedup vs baseline: 1.0184x; 1.0137x over previous
import jax
import jax.numpy as jnp
from jax import lax
from jax.experimental import pallas as pl
from jax.experimental.pallas import tpu as pltpu
from jax.experimental.pallas import tpu_sc as plsc

VOCAB = 100000
D_MODEL = 1024
BATCH = 4
SEQ_LEN = 2048

NUM_CORES = 2
NUM_SUBCORES = 16
NW = NUM_CORES * NUM_SUBCORES
B_TOTAL = BATCH * SEQ_LEN
B_PER_W = B_TOTAL // NW
W_PER_B = NW // BATCH
CHUNK = 32
NCHUNK = B_PER_W // CHUNK
NBUF = 2
NLOOP = NCHUNK // NBUF


def _emb_body(tok_hbm, table_hbm, out_hbm, idx_v, *rest):
    bufs = rest[:NBUF]
    gsem, ssem = rest[NBUF:]
    wid = lax.axis_index("s") * NUM_CORES + lax.axis_index("c")
    b = wid // W_PER_B
    off = (wid % W_PER_B) * B_PER_W
    pltpu.sync_copy(tok_hbm.at[b, pl.ds(off, B_PER_W)], idx_v)

    def gather_start(c, bi):
        pltpu.make_async_copy(
            table_hbm.at[idx_v.at[pl.ds(c * CHUNK, CHUNK)]], bufs[bi], gsem).start()

    def gather_wait(bi):
        pltpu.make_async_copy(
            table_hbm.at[idx_v.at[pl.ds(0, CHUNK)]], bufs[bi], gsem).wait()

    def scatter_start(c, bi):
        pltpu.make_async_copy(
            bufs[bi], out_hbm.at[b, pl.ds(off + c * CHUNK, CHUNK)], ssem).start()

    def scatter_wait(bi):
        pltpu.make_async_copy(
            bufs[bi], out_hbm.at[b, pl.ds(off, CHUNK)], ssem).wait()

    for bi in range(NBUF):
        gather_start(bi, bi)

    @pl.loop(0, NLOOP)
    def _outer(g):
        for bi in range(NBUF):
            c = g * NBUF + bi
            gather_wait(bi)
            scatter_start(c, bi)

            @pl.when(g < NLOOP - 1)
            def _refill():
                scatter_wait(bi)
                gather_start(c + NBUF, bi)

    for bi in range(NBUF):
        scatter_wait(bi)


@jax.jit
def _embedding_lookup(tokens, table):
    mesh = plsc.VectorSubcoreMesh(core_axis_name="c", subcore_axis_name="s")
    run = pl.kernel(
        _emb_body,
        out_type=jax.ShapeDtypeStruct((BATCH, SEQ_LEN, D_MODEL), jnp.float32),
        mesh=mesh,
        scratch_types=(
            [pltpu.VMEM((B_PER_W,), jnp.int32)]
            + [pltpu.VMEM((CHUNK, D_MODEL), jnp.float32) for _ in range(NBUF)]
            + [pltpu.SemaphoreType.DMA, pltpu.SemaphoreType.DMA]
        ),
    )
    return run(tokens, table)


def kernel(tokens, embedding_weight):
    return _embedding_lookup(tokens.astype(jnp.int32), embedding_weight)
